# baseline (speedup 1.0000x reference)
import functools

import jax
import jax.numpy as jnp
from jax import lax
from jax.experimental import pallas as pl
from jax.experimental.pallas import tpu as pltpu
from jax.experimental.pallas import tpu_sc as plsc

_S = 32
_D = 64
_N = 1025
_RPW = 32
_DV = _D // 16


def _t0(tv_v, th_v, d):
    return tv_v[0, pl.ds(16 * d, 16)] + th_v[0, pl.ds(16 * d, 16)]


def _fill_a(tv_v, th_v, buf, i):
    gm = i - 1
    bi = gm // _S
    ci = gm - bi * _S
    h0 = 33 - ci
    for d in range(_DV):
        buf[0, pl.ds(16 * d, 16)] = _t0(tv_v, th_v, d)

    def b_body(b, _):
        vrow = 33 - bi + b
        base = 1 + _S * b
        for c in range(_S):
            for d in range(_DV):
                v = tv_v[vrow, pl.ds(16 * d, 16)] + th_v[h0 + c, pl.ds(16 * d, 16)]
                buf[base + c, pl.ds(16 * d, 16)] = v
        return 0

    lax.fori_loop(0, 15, b_body, 0, unroll=False)
    vrow = 33 - bi + 15
    for c in range(31):
        for d in range(_DV):
            v = tv_v[vrow, pl.ds(16 * d, 16)] + th_v[h0 + c, pl.ds(16 * d, 16)]
            buf[481 + c, pl.ds(16 * d, 16)] = v


def _fill_b(tv_v, th_v, buf, i):
    gm = i - 1
    bi = gm // _S
    ci = gm - bi * _S
    h0 = 33 - ci
    vrow15 = 33 - bi + 15
    for d in range(_DV):
        v = tv_v[vrow15, pl.ds(16 * d, 16)] + th_v[h0 + 31, pl.ds(16 * d, 16)]
        buf[0, pl.ds(16 * d, 16)] = v

    def b_body(b, _):
        vrow = 33 - bi + b
        base = 1 + _S * (b - 16)
        for c in range(_S):
            for d in range(_DV):
                v = tv_v[vrow, pl.ds(16 * d, 16)] + th_v[h0 + c, pl.ds(16 * d, 16)]
                buf[base + c, pl.ds(16 * d, 16)] = v
        return 0

    lax.fori_loop(16, 31, b_body, 0, unroll=False)
    vrow31 = 33 - bi + 31
    for c in range(_S):
        for d in range(_DV):
            v = tv_v[vrow31, pl.ds(16 * d, 16)] + th_v[h0 + c, pl.ds(16 * d, 16)]
            buf[481 + c, pl.ds(16 * d, 16)] = v


def _fill_pad(tv_v, th_v, buf, n):
    t0 = [_t0(tv_v, th_v, d) for d in range(_DV)]

    def j_body(j, _):
        for d in range(_DV):
            buf[j, pl.ds(16 * d, 16)] = t0[d]
        return 0

    lax.fori_loop(0, n, j_body, 0, unroll=False)


def _do_row(tv_v, th_v, buf, out_hbm, i):
    @pl.when(i > 0)
    def _():
        _fill_a(tv_v, th_v, buf, i)

    @pl.when(i == 0)
    def _():
        _fill_pad(tv_v, th_v, buf, 513)

    pltpu.sync_copy(buf.at[pl.ds(0, 512), :], out_hbm.at[i, pl.ds(0, 512)])

    @pl.when(i > 0)
    def _():
        _fill_b(tv_v, th_v, buf, i)

    pltpu.sync_copy(buf.at[pl.ds(0, 504), :], out_hbm.at[i, pl.ds(512, 504)])
    pltpu.sync_copy(buf.at[pl.ds(504, 8), :], out_hbm.at[i, pl.ds(1016, 8)])
    pltpu.sync_copy(buf.at[pl.ds(512, 1), :], out_hbm.at[i, pl.ds(1024, 1)])


def _sc_body(tv_hbm, th_hbm, out_hbm, tv_v, th_v, buf):
    wid = lax.axis_index("s") * 2 + lax.axis_index("c")
    pltpu.sync_copy(tv_hbm, tv_v)
    pltpu.sync_copy(th_hbm, th_v)

    def row_body(r, _):
        _do_row(tv_v, th_v, buf, out_hbm, wid * _RPW + r)
        return 0

    lax.fori_loop(0, _RPW, row_body, 0, unroll=False)

    @pl.when(wid == 0)
    def _():
        _do_row(tv_v, th_v, buf, out_hbm, _N - 1)


def kernel(length_q, length_k, embeddings_table_v, embeddings_table_h):
    del length_q, length_k
    run = functools.partial(
        pl.kernel,
        out_type=jax.ShapeDtypeStruct((_N, _N, _D), jnp.float32),
        mesh=plsc.VectorSubcoreMesh(core_axis_name="c", subcore_axis_name="s"),
        scratch_types=[
            pltpu.VMEM((66, _D), jnp.float32),
            pltpu.VMEM((66, _D), jnp.float32),
            pltpu.VMEM((520, _D), jnp.float32),
        ],
    )(_sc_body)
    return run(embeddings_table_v, embeddings_table_h)


# speedup vs baseline: 1.1392x; 1.1392x over previous
import functools

import jax
import jax.numpy as jnp
from jax import lax
from jax.experimental import pallas as pl
from jax.experimental.pallas import tpu as pltpu
from jax.experimental.pallas import tpu_sc as plsc

_S = 32
_D = 64
_N = 1025
_RPW = 32
_DV = _D // 16
_QL = 256


def _t0(tv_v, th_v, d):
    return tv_v[0, pl.ds(16 * d, 16)] + th_v[0, pl.ds(16 * d, 16)]


def _fill_quarter(tv_v, th_v, buf, i, k):
    gm = i - 1
    bi = gm // _S
    ci = gm - bi * _S
    h0 = 33 - ci
    v0 = 33 - bi
    if k == 0:
        for d in range(_DV):
            buf[0, pl.ds(16 * d, 16)] = _t0(tv_v, th_v, d)
    else:
        for d in range(_DV):
            v = tv_v[v0 + (8 * k - 1), pl.ds(16 * d, 16)] \
                + th_v[h0 + 31, pl.ds(16 * d, 16)]
            buf[0, pl.ds(16 * d, 16)] = v

    def block_cols(bb, c_hi):
        vrow = v0 + 8 * k + bb
        vv = [tv_v[vrow, pl.ds(16 * d, 16)] for d in range(_DV)]
        base = 1 + _S * bb

        def c_body(c, _):
            for d in range(_DV):
                buf[base + c, pl.ds(16 * d, 16)] = \
                    vv[d] + th_v[h0 + c, pl.ds(16 * d, 16)]
            return 0

        lax.fori_loop(0, c_hi, c_body, 0, unroll=False)

    def b_body(bb, _):
        block_cols(bb, _S)
        return 0

    lax.fori_loop(0, 7, b_body, 0, unroll=False)
    block_cols(7, 31)
    if k == 3:
        for d in range(_DV):
            v = tv_v[v0 + 31, pl.ds(16 * d, 16)] + th_v[h0 + 31, pl.ds(16 * d, 16)]
            buf[_QL, pl.ds(16 * d, 16)] = v


def _fill_pad(tv_v, th_v, buf):
    t0 = [_t0(tv_v, th_v, d) for d in range(_DV)]

    def j_body(j, _):
        for d in range(_DV):
            buf[j, pl.ds(16 * d, 16)] = t0[d]
        return 0

    lax.fori_loop(0, _QL + 1, j_body, 0, unroll=False)


def _wait_q(buf, sem, out_hbm, k):
    pltpu.make_async_copy(
        buf.at[pl.ds(0, _QL), :], out_hbm.at[0, pl.ds(0, _QL)], sem).wait()
    if k == 3:
        pltpu.make_async_copy(
            buf.at[pl.ds(_QL, 1), :], out_hbm.at[0, pl.ds(1024, 1)], sem).wait()


def _do_row(tv_v, th_v, bufs, sems, out_hbm, i, first):
    for k in range(4):
        buf = bufs[k % 2]
        sem = sems[k % 2]
        prev_k = k - 2 if k >= 2 else k + 2

        @pl.when(jnp.logical_not(first) | (k >= 2))
        def _(buf=buf, sem=sem, prev_k=prev_k):
            _wait_q(buf, sem, out_hbm, prev_k)

        @pl.when(i > 0)
        def _(buf=buf, k=k):
            _fill_quarter(tv_v, th_v, buf, i, k)

        @pl.when(i == 0)
        def _(buf=buf):
            _fill_pad(tv_v, th_v, buf)

        pltpu.make_async_copy(
            buf.at[pl.ds(0, _QL), :],
            out_hbm.at[i, pl.ds(_QL * k, _QL)], sem).start()
        if k == 3:
            pltpu.make_async_copy(
                buf.at[pl.ds(_QL, 1), :],
                out_hbm.at[i, pl.ds(1024, 1)], sem).start()


def _sc_body(tv_hbm, th_hbm, out_hbm, tv_v, th_v, buf0, buf1, sem0, sem1):
    wid = lax.axis_index("s") * 2 + lax.axis_index("c")
    pltpu.sync_copy(tv_hbm, tv_v)
    pltpu.sync_copy(th_hbm, th_v)
    bufs = (buf0, buf1)
    sems = (sem0, sem1)

    def row_body(r, _):
        _do_row(tv_v, th_v, bufs, sems, out_hbm, wid * _RPW + r, r == 0)
        return 0

    lax.fori_loop(0, _RPW, row_body, 0, unroll=False)

    @pl.when(wid == 0)
    def _():
        _do_row(tv_v, th_v, bufs, sems, out_hbm, _N - 1,
                jnp.bool_(False))

    _wait_q(buf0, sem0, out_hbm, 2)
    _wait_q(buf1, sem1, out_hbm, 3)


def kernel(length_q, length_k, embeddings_table_v, embeddings_table_h):
    del length_q, length_k
    run = functools.partial(
        pl.kernel,
        out_type=jax.ShapeDtypeStruct((_N, _N, _D), jnp.float32),
        mesh=plsc.VectorSubcoreMesh(core_axis_name="c", subcore_axis_name="s"),
        scratch_types=[
            pltpu.VMEM((66, _D), jnp.float32),
            pltpu.VMEM((66, _D), jnp.float32),
            pltpu.VMEM((_QL + 8, _D), jnp.float32),
            pltpu.VMEM((_QL + 8, _D), jnp.float32),
            pltpu.SemaphoreType.DMA,
            pltpu.SemaphoreType.DMA,
        ],
    )(_sc_body)
    return run(embeddings_table_v, embeddings_table_h)


# speedup vs baseline: 2.2914x; 2.0115x over previous
import jax
import jax.numpy as jnp
from jax.experimental import pallas as pl
from jax.experimental.pallas import tpu as pltpu

_S = 32
_D = 64
_N = 1025
_R = 5
_STEPS = _N // _R
_NBUF = 8


def _compute_rows(tv_ref, th_ref, buf, s):
    t0 = tv_ref[0:1, :] + th_ref[0:1, :]
    for r in range(_R):
        g = s * _R + r
        gm = jnp.maximum(g - 1, 0)
        bi = gm // _S
        ci = gm - bi * _S
        vs = tv_ref[pl.ds(33 - bi, _S), :]
        hs = th_ref[pl.ds(33 - ci, _S), :]
        pat = (vs[:, None, :] + hs[None, :, :]).reshape(_S * _S, _D)
        buf[r, 0:1, :] = t0
        buf[r, 1:, :] = pat
    @pl.when(s == 0)
    def _():
        buf[0, :, :] = jnp.broadcast_to(t0, (_N, _D))


def _rp2d_body(tv_ref, th_ref, out_ref, *scratch):
    bufs = scratch[:_NBUF]
    sems = scratch[_NBUF:]
    s = pl.program_id(0)
    for b in range(_NBUF):
        @pl.when(s % _NBUF == b)
        def _(b=b):
            @pl.when(s >= _NBUF)
            def _():
                pltpu.make_async_copy(
                    bufs[b], out_ref.at[pl.ds(0, _R)], sems[b]).wait()
            _compute_rows(tv_ref, th_ref, bufs[b], s)
            pltpu.make_async_copy(
                bufs[b], out_ref.at[pl.ds(s * _R, _R)], sems[b]).start()
    @pl.when(s == _STEPS - 1)
    def _():
        for b in range(_NBUF):
            pltpu.make_async_copy(
                bufs[b], out_ref.at[pl.ds(0, _R)], sems[b]).wait()


def kernel(length_q, length_k, embeddings_table_v, embeddings_table_h):
    del length_q, length_k
    tv = jnp.pad(embeddings_table_v, ((0, 6), (0, 0)))
    th = jnp.pad(embeddings_table_h, ((0, 6), (0, 0)))
    return pl.pallas_call(
        _rp2d_body,
        grid=(_STEPS,),
        in_specs=[
            pl.BlockSpec((72, _D), lambda i: (0, 0)),
            pl.BlockSpec((72, _D), lambda i: (0, 0)),
        ],
        out_specs=pl.BlockSpec(memory_space=pl.ANY),
        out_shape=jax.ShapeDtypeStruct((_N, _N, _D), jnp.float32),
        scratch_shapes=(
            [pltpu.VMEM((_R, _N, _D), jnp.float32) for _ in range(_NBUF)]
            + [pltpu.SemaphoreType.DMA for _ in range(_NBUF)]
        ),
    )(tv, th)


# speedup vs baseline: 2.2924x; 1.0004x over previous
import jax
import jax.numpy as jnp
from jax.experimental import pallas as pl
from jax.experimental.pallas import tpu as pltpu

_S = 32
_D = 64
_N = 1025
_R = 5
_STEPS = _N // _R
_NBUF = 4


def _compute_rows(tv_ref, th_ref, buf, s):
    t0 = tv_ref[0:1, :] + th_ref[0:1, :]
    for r in range(_R):
        g = s * _R + r
        gm = jnp.maximum(g - 1, 0)
        bi = gm // _S
        ci = gm - bi * _S
        vs = tv_ref[pl.ds(33 - bi, _S), :]
        hs = th_ref[pl.ds(33 - ci, _S), :]
        pat = (vs[:, None, :] + hs[None, :, :]).reshape(_S * _S, _D)
        buf[r, 0:1, :] = t0
        buf[r, 1:, :] = pat
    @pl.when(s == 0)
    def _():
        buf[0, :, :] = jnp.broadcast_to(t0, (_N, _D))


def _rp2d_body(tv_ref, th_ref, out_ref, *scratch):
    bufs = scratch[:_NBUF]
    sems = scratch[_NBUF:]
    s = pl.program_id(0)
    for b in range(_NBUF):
        @pl.when(s % _NBUF == b)
        def _(b=b):
            @pl.when(s >= _NBUF)
            def _():
                pltpu.make_async_copy(
                    bufs[b], out_ref.at[pl.ds(0, _R)], sems[b]).wait()
            _compute_rows(tv_ref, th_ref, bufs[b], s)
            pltpu.make_async_copy(
                bufs[b], out_ref.at[pl.ds(s * _R, _R)], sems[b]).start()
    @pl.when(s == _STEPS - 1)
    def _():
        for b in range(_NBUF):
            pltpu.make_async_copy(
                bufs[b], out_ref.at[pl.ds(0, _R)], sems[b]).wait()


def kernel(length_q, length_k, embeddings_table_v, embeddings_table_h):
    del length_q, length_k
    tv = jnp.pad(embeddings_table_v, ((0, 6), (0, 0)))
    th = jnp.pad(embeddings_table_h, ((0, 6), (0, 0)))
    return pl.pallas_call(
        _rp2d_body,
        grid=(_STEPS,),
        in_specs=[
            pl.BlockSpec((72, _D), lambda i: (0, 0)),
            pl.BlockSpec((72, _D), lambda i: (0, 0)),
        ],
        out_specs=pl.BlockSpec(memory_space=pl.ANY),
        out_shape=jax.ShapeDtypeStruct((_N, _N, _D), jnp.float32),
        scratch_shapes=(
            [pltpu.VMEM((_R, _N, _D), jnp.float32) for _ in range(_NBUF)]
            + [pltpu.SemaphoreType.DMA for _ in range(_NBUF)]
        ),
    )(tv, th)
